# baseline (speedup 1.0000x reference)
import functools
import jax
import jax.numpy as jnp
from jax import lax
from jax.experimental import pallas as pl
from jax.experimental.pallas import tpu as pltpu, tpu_sc as plsc

_B, _S, _D = 4, 4096, 1024
_NC, _NS = 2, 16
_NW = _NC * _NS
_SW = _S // _NW
_R = 32
_CH = _SW // _R
_CW = _R * _D


def _sc_body(x_hbm, pos_hbm, out_hbm, xbuf, pbuf):
    wid = lax.axis_index("s") * _NC + lax.axis_index("c")
    s0 = wid * _SW
    for c in range(_CH):
        row = s0 + c * _R
        pltpu.sync_copy(pos_hbm.at[pl.ds(row * _D, _CW)], pbuf)
        for b in range(_B):
            off = (b * _S + row) * _D
            pltpu.sync_copy(x_hbm.at[pl.ds(off, _CW)], xbuf)

            @plsc.parallel_loop(0, _CW, step=16, unroll=8)
            def _add(i):
                xbuf[pl.ds(i, 16)] = xbuf[pl.ds(i, 16)] + pbuf[pl.ds(i, 16)]

            pltpu.sync_copy(xbuf, out_hbm.at[pl.ds(off, _CW)])


def kernel(x, pos_table):
    B, S, D = x.shape
    run = pl.kernel(
        _sc_body,
        out_type=jax.ShapeDtypeStruct((B * S * D,), jnp.float32),
        mesh=plsc.VectorSubcoreMesh(core_axis_name="c", subcore_axis_name="s"),
        scratch_types=[
            pltpu.VMEM((_CW,), jnp.float32),
            pltpu.VMEM((_CW,), jnp.float32),
        ],
    )
    out = run(x.reshape(-1), pos_table.reshape(-1))
    return out.reshape(B, S, D)


# speedup vs baseline: 1.1429x; 1.1429x over previous
import jax
import jax.numpy as jnp
from jax import lax
from jax.experimental import pallas as pl
from jax.experimental.pallas import tpu as pltpu, tpu_sc as plsc

_B, _S, _D = 4, 4096, 1024
_NC, _NS = 2, 16
_NW = _NC * _NS
_SW = _S // _NW
_R = 16
_CH = _SW // _R
_CW = _R * _D
_NITEMS = _CH * _B


def _sc_body(x_hbm, pos_hbm, out_hbm,
             xb0, xb1, pb0, pb1,
             ls0, ls1, ss0, ss1, ps0, ps1):
    xb, pb = [xb0, xb1], [pb0, pb1]
    ls, ss, ps = [ls0, ls1], [ss0, ss1], [ps0, ps1]
    wid = lax.axis_index("s") * _NC + lax.axis_index("c")
    s0 = wid * _SW

    def xoff(k):
        c, b = divmod(k, _B)
        return (b * _S + s0 + c * _R) * _D

    def poff(c):
        return (s0 + c * _R) * _D

    pltpu.async_copy(pos_hbm.at[pl.ds(poff(0), _CW)], pb[0], ps[0])
    pltpu.async_copy(x_hbm.at[pl.ds(xoff(0), _CW)], xb[0], ls[0])
    pending_store = [None, None]

    for k in range(_NITEMS):
        bi = k % 2
        c = k // _B
        if k + 1 < _NITEMS:
            nb = (k + 1) % 2
            if pending_store[nb] is not None:
                pending_store[nb].wait()
                pending_store[nb] = None
            pltpu.async_copy(x_hbm.at[pl.ds(xoff(k + 1), _CW)], xb[nb], ls[nb])
        if k % _B == 0 and c + 1 < _CH:
            pltpu.async_copy(
                pos_hbm.at[pl.ds(poff(c + 1), _CW)], pb[(c + 1) % 2],
                ps[(c + 1) % 2])
        pltpu.make_async_copy(
            x_hbm.at[pl.ds(xoff(k), _CW)], xb[bi], ls[bi]).wait()
        if k % _B == 0:
            pltpu.make_async_copy(
                pos_hbm.at[pl.ds(poff(c), _CW)], pb[c % 2], ps[c % 2]).wait()

        xc, pc = xb[bi], pb[c % 2]

        @plsc.parallel_loop(0, _CW, step=16, unroll=8)
        def _add(i):
            xc[pl.ds(i, 16)] = xc[pl.ds(i, 16)] + pc[pl.ds(i, 16)]

        pending_store[bi] = pltpu.async_copy(
            xc, out_hbm.at[pl.ds(xoff(k), _CW)], ss[bi])

    for h in pending_store:
        if h is not None:
            h.wait()


def kernel(x, pos_table):
    B, S, D = x.shape
    run = pl.kernel(
        _sc_body,
        out_type=jax.ShapeDtypeStruct((B * S * D,), jnp.float32),
        mesh=plsc.VectorSubcoreMesh(core_axis_name="c", subcore_axis_name="s"),
        scratch_types=(
            [pltpu.VMEM((_CW,), jnp.float32)] * 4
            + [pltpu.SemaphoreType.DMA] * 6
        ),
    )
    out = run(x.reshape(-1), pos_table.reshape(-1))
    return out.reshape(B, S, D)


# speedup vs baseline: 2.8762x; 2.5165x over previous
import jax
import jax.numpy as jnp
from jax import lax
from jax.experimental import pallas as pl
from jax.experimental.pallas import tpu as pltpu, tpu_sc as plsc

_B, _S, _D = 4, 4096, 1024
_NC, _NS = 2, 16
_NW = _NC * _NS
_SW = _S // _NW
_R = 16
_CH = _SW // _R
_NITEMS = _CH * _B


def _sc_body(x_hbm, pos_hbm, out_hbm,
             xb0, xb1, pb0, pb1,
             ls0, ls1, ss0, ss1, ps0, ps1):
    xb, pb = [xb0, xb1], [pb0, pb1]
    ls, ss, ps = [ls0, ls1], [ss0, ss1], [ps0, ps1]
    wid = lax.axis_index("s") * _NC + lax.axis_index("c")
    s0 = wid * _SW

    def item(k):
        c, b = divmod(k, _B)
        return b, s0 + c * _R

    pltpu.async_copy(pos_hbm.at[pl.ds(s0, _R), :], pb[0], ps[0])
    b0, r0 = item(0)
    pltpu.async_copy(x_hbm.at[b0, pl.ds(r0, _R), :], xb[0], ls[0])
    pending_store = [None, None]

    for k in range(_NITEMS):
        bi = k % 2
        c = k // _B
        b, row = item(k)
        if k + 1 < _NITEMS:
            nb = (k + 1) % 2
            if pending_store[nb] is not None:
                pending_store[nb].wait()
                pending_store[nb] = None
            bn, rn = item(k + 1)
            pltpu.async_copy(x_hbm.at[bn, pl.ds(rn, _R), :], xb[nb], ls[nb])
        if k % _B == 0 and c + 1 < _CH:
            pltpu.async_copy(
                pos_hbm.at[pl.ds(s0 + (c + 1) * _R, _R), :],
                pb[(c + 1) % 2], ps[(c + 1) % 2])
        pltpu.make_async_copy(
            x_hbm.at[b, pl.ds(row, _R), :], xb[bi], ls[bi]).wait()
        if k % _B == 0:
            pltpu.make_async_copy(
                pos_hbm.at[pl.ds(s0 + c * _R, _R), :], pb[c % 2],
                ps[c % 2]).wait()

        xc, pc = xb[bi], pb[c % 2]

        @plsc.parallel_loop(0, _D, step=16)
        def _add(i):
            for r in range(_R):
                xc[r, pl.ds(i, 16)] = xc[r, pl.ds(i, 16)] + pc[r, pl.ds(i, 16)]

        pending_store[bi] = pltpu.async_copy(
            xc, out_hbm.at[b, pl.ds(row, _R), :], ss[bi])

    for h in pending_store:
        if h is not None:
            h.wait()


def kernel(x, pos_table):
    B, S, D = x.shape
    run = pl.kernel(
        _sc_body,
        out_type=jax.ShapeDtypeStruct((B, S, D), jnp.float32),
        mesh=plsc.VectorSubcoreMesh(core_axis_name="c", subcore_axis_name="s"),
        scratch_types=(
            [pltpu.VMEM((_R, _D), jnp.float32)] * 4
            + [pltpu.SemaphoreType.DMA] * 6
        ),
        compiler_params=pltpu.CompilerParams(use_tc_tiling_on_sc=True),
    )
    return run(x, pos_table)
